# Initial kernel scaffold; baseline (speedup 1.0000x reference)
import jax
import jax.numpy as jnp
from jax.experimental import pallas as pl


def kernel(x, edge_index, self_weight, neighbor_weight, bias):
    raise NotImplementedError("write your pallas kernel here")


# speedup vs baseline: 10.4415x; 10.4415x over previous
import functools

import jax
import jax.numpy as jnp
from jax import lax
from jax.experimental import pallas as pl
from jax.experimental.pallas import tpu as pltpu
from jax.experimental.pallas import tpu_sc as plsc

N = 10000
E = 320000
D = 128
NPAD = 10240
NC = 2
NS = 16
ER = E // D
ERP = 2560
CH = 128

_MESH = plsc.VectorSubcoreMesh(
    core_axis_name="c", subcore_axis_name="s", num_cores=NC, num_subcores=NS)

_DEG_ROWS = ERP // NS
_NODES_PER_SUB = NPAD // (NC * NS)


def _sc_deg_body(row2d, dis128, idx_v, ones_v, zro_v, dstage_v, drows_v,
                 deg_sh, sem):
    c = lax.axis_index("c")
    s = lax.axis_index("s")

    z16 = jnp.zeros((16,), jnp.float32)
    for k in range(640 // 16):
        zro_v[pl.ds(k * 16, 16)] = z16
    o16 = jnp.ones((16,), jnp.float32)
    for k in range(CH // 16):
        ones_v[pl.ds(k * 16, 16)] = o16

    pltpu.sync_copy(zro_v, deg_sh.at[pl.ds(s * 640, 640)])

    pltpu.sync_copy(row2d.at[pl.ds(s * _DEG_ROWS, _DEG_ROWS), :], idx_v)

    plsc.subcore_barrier()

    def _scatter(j, carry):
        pltpu.sync_copy(ones_v, deg_sh.at[idx_v.at[j]], add=True)
        return carry

    lax.fori_loop(0, _DEG_ROWS, _scatter, 0, unroll=False)

    plsc.subcore_barrier()

    nbase = (c * NS + s) * _NODES_PER_SUB
    pltpu.sync_copy(deg_sh.at[pl.ds(nbase, _NODES_PER_SUB)], dstage_v)

    for v in range(_NODES_PER_SUB // 16):
        d = dstage_v[pl.ds(v * 16, 16)]
        i = lax.bitcast_convert_type(d, jnp.int32)
        i = jnp.int32(0x5F3759DF) - lax.shift_right_logical(i, 1)
        y = lax.bitcast_convert_type(i, jnp.float32)
        half = d * jnp.float32(0.5)
        for _ in range(3):
            y = y * (jnp.float32(1.5) - half * y * y)
        dstage_v[pl.ds(v * 16, 16)] = y

    def _splat(g, carry):
        v = dstage_v[pl.ds(g * 16, 16)]
        for r in range(16):
            v16 = lax.broadcast(v[r], (16,))
            for cc in range(D // 16):
                drows_v[r, pl.ds(cc * 16, 16)] = v16
        pltpu.sync_copy(drows_v, dis128.at[pl.ds(nbase + g * 16, 16), :])
        return carry

    lax.fori_loop(0, _NODES_PER_SUB // 16, _splat, 0, unroll=False)


_sc_deg = functools.partial(
    pl.kernel,
    out_type=jax.ShapeDtypeStruct((NPAD, D), jnp.float32),
    mesh=_MESH,
    scratch_types=[
        pltpu.VMEM((_DEG_ROWS, CH), jnp.int32),
        pltpu.VMEM((CH,), jnp.float32),
        pltpu.VMEM((640,), jnp.float32),
        pltpu.VMEM((_NODES_PER_SUB,), jnp.float32),
        pltpu.VMEM((16, D), jnp.float32),
        pltpu.VMEM_SHARED((NPAD,), jnp.float32),
        pltpu.SemaphoreType.DMA,
    ],
)(_sc_deg_body)


def _tc_main_body(x_ref, ws_ref, wn_ref, b_ref, dis_ref, self_ref, hp_ref):
    x = x_ref[...]
    self_ref[...] = (
        jnp.dot(x, ws_ref[...], preferred_element_type=jnp.float32)
        + b_ref[...])
    hp_ref[...] = dis_ref[...] * jnp.dot(
        x, wn_ref[...], preferred_element_type=jnp.float32)


_tc_main = pl.pallas_call(
    _tc_main_body,
    out_shape=(
        jax.ShapeDtypeStruct((N, D), jnp.float32),
        jax.ShapeDtypeStruct((N, D), jnp.float32),
    ),
)


_AGG_ROWS = ERP // (NC * NS)


def _sc_agg_body(hp, col2d, row2d, q, cidx_v, ridx_v, rows_v, zblk_v,
                 acc_sh, sem):
    c = lax.axis_index("c")
    s = lax.axis_index("s")
    w = c * NS + s

    z16 = jnp.zeros((16,), jnp.float32)
    for r in range(16):
        for cc in range(D // 16):
            zblk_v[r, pl.ds(cc * 16, 16)] = z16

    def _zero(k, carry):
        pltpu.sync_copy(zblk_v, acc_sh.at[pl.ds(s * 640 + k * 16, 16), :])
        return carry

    lax.fori_loop(0, 40, _zero, 0, unroll=False)

    b0 = w * _AGG_ROWS
    pltpu.sync_copy(col2d.at[pl.ds(b0, _AGG_ROWS), :], cidx_v)
    pltpu.sync_copy(row2d.at[pl.ds(b0, _AGG_ROWS), :], ridx_v)

    plsc.subcore_barrier()

    def _agg(j, carry):
        pltpu.async_copy(hp.at[cidx_v.at[j]], rows_v, sem).wait()
        pltpu.sync_copy(rows_v, acc_sh.at[ridx_v.at[j]], add=True)
        return carry

    lax.fori_loop(0, _AGG_ROWS, _agg, 0, unroll=False)

    plsc.subcore_barrier()

    pltpu.sync_copy(acc_sh.at[pl.ds(s * 640, 640), :],
                    q.at[pl.ds(c * NPAD + s * 640, 640), :])


_sc_agg = functools.partial(
    pl.kernel,
    out_type=jax.ShapeDtypeStruct((NC * NPAD, D), jnp.float32),
    mesh=_MESH,
    scratch_types=[
        pltpu.VMEM((_AGG_ROWS, CH), jnp.int32),
        pltpu.VMEM((_AGG_ROWS, CH), jnp.int32),
        pltpu.VMEM((CH, D), jnp.float32),
        pltpu.VMEM((16, D), jnp.float32),
        pltpu.VMEM_SHARED((NPAD, D), jnp.float32),
        pltpu.SemaphoreType.DMA,
    ],
)(_sc_agg_body)


def _tc_comb_body(self_ref, dis_ref, q0_ref, q1_ref, o_ref):
    o_ref[...] = self_ref[...] + dis_ref[...] * (q0_ref[...] + q1_ref[...])


_tc_comb = pl.pallas_call(
    _tc_comb_body,
    out_shape=jax.ShapeDtypeStruct((N, D), jnp.float32),
)


def kernel(x, edge_index, self_weight, neighbor_weight, bias):
    npad_edges = ERP * D - E
    row_pad = jnp.concatenate(
        [edge_index[0], jnp.full((npad_edges,), NPAD - 1, jnp.int32)])
    col_pad = jnp.concatenate(
        [edge_index[1], jnp.zeros((npad_edges,), jnp.int32)])
    row2d = row_pad.reshape(ERP, D)
    col2d = col_pad.reshape(ERP, D)
    dis128 = _sc_deg(row2d)
    dis_n = dis128[:N]
    selfs, hp = _tc_main(x, self_weight, neighbor_weight,
                         bias.reshape(1, D), dis_n)
    q = _sc_agg(hp, col2d, row2d)
    return _tc_comb(selfs, dis_n, q[:N], q[NPAD:NPAD + N])


# speedup vs baseline: 11.9286x; 1.1424x over previous
import functools

import jax
import jax.numpy as jnp
from jax import lax
from jax.experimental import pallas as pl
from jax.experimental.pallas import tpu as pltpu
from jax.experimental.pallas import tpu_sc as plsc

N = 10000
E = 320000
D = 128
NPAD = 10240
NC = 2
NS = 16
ER = E // D
ERP = 2560
CH = 128

_MESH = plsc.VectorSubcoreMesh(
    core_axis_name="c", subcore_axis_name="s", num_cores=NC, num_subcores=NS)

_DEG_ROWS = ERP // NS
_NODES_PER_SUB = NPAD // (NC * NS)


def _sc_deg_body(row2d, dis128, idx_v, ones_v, zro_v, dstage_v, drows_v,
                 deg_sh, sem):
    c = lax.axis_index("c")
    s = lax.axis_index("s")

    z16 = jnp.zeros((16,), jnp.float32)
    for k in range(640 // 16):
        zro_v[pl.ds(k * 16, 16)] = z16
    o16 = jnp.ones((16,), jnp.float32)
    for k in range(CH // 16):
        ones_v[pl.ds(k * 16, 16)] = o16

    pltpu.sync_copy(zro_v, deg_sh.at[pl.ds(s * 640, 640)])

    pltpu.sync_copy(row2d.at[pl.ds(s * _DEG_ROWS, _DEG_ROWS), :], idx_v)

    plsc.subcore_barrier()

    def _scatter(j, carry):
        pltpu.sync_copy(ones_v, deg_sh.at[idx_v.at[j]], add=True)
        return carry

    lax.fori_loop(0, _DEG_ROWS, _scatter, 0, unroll=False)

    plsc.subcore_barrier()

    nbase = (c * NS + s) * _NODES_PER_SUB
    pltpu.sync_copy(deg_sh.at[pl.ds(nbase, _NODES_PER_SUB)], dstage_v)

    for v in range(_NODES_PER_SUB // 16):
        d = dstage_v[pl.ds(v * 16, 16)]
        i = lax.bitcast_convert_type(d, jnp.int32)
        i = jnp.int32(0x5F3759DF) - lax.shift_right_logical(i, 1)
        y = lax.bitcast_convert_type(i, jnp.float32)
        half = d * jnp.float32(0.5)
        for _ in range(3):
            y = y * (jnp.float32(1.5) - half * y * y)
        dstage_v[pl.ds(v * 16, 16)] = y

    def _splat(g, carry):
        v = dstage_v[pl.ds(g * 16, 16)]
        for r in range(16):
            v16 = lax.broadcast(v[r], (16,))
            for cc in range(D // 16):
                drows_v[r, pl.ds(cc * 16, 16)] = v16
        pltpu.sync_copy(drows_v, dis128.at[pl.ds(nbase + g * 16, 16), :])
        return carry

    lax.fori_loop(0, _NODES_PER_SUB // 16, _splat, 0, unroll=False)


_sc_deg = functools.partial(
    pl.kernel,
    out_type=jax.ShapeDtypeStruct((NPAD, D), jnp.float32),
    mesh=_MESH,
    scratch_types=[
        pltpu.VMEM((_DEG_ROWS, CH), jnp.int32),
        pltpu.VMEM((CH,), jnp.float32),
        pltpu.VMEM((640,), jnp.float32),
        pltpu.VMEM((_NODES_PER_SUB,), jnp.float32),
        pltpu.VMEM((16, D), jnp.float32),
        pltpu.VMEM_SHARED((NPAD,), jnp.float32),
        pltpu.SemaphoreType.DMA,
    ],
)(_sc_deg_body)


def _tc_main_body(x_ref, ws_ref, wn_ref, b_ref, dis_ref, self_ref, hp_ref):
    x = x_ref[...]
    self_ref[...] = (
        jnp.dot(x, ws_ref[...], preferred_element_type=jnp.float32)
        + b_ref[...])
    hp_ref[...] = dis_ref[...] * jnp.dot(
        x, wn_ref[...], preferred_element_type=jnp.float32)


_tc_main = pl.pallas_call(
    _tc_main_body,
    out_shape=(
        jax.ShapeDtypeStruct((N, D), jnp.float32),
        jax.ShapeDtypeStruct((N, D), jnp.float32),
    ),
)


_AGG_ROWS = ERP // (NC * NS)
_HALF = _AGG_ROWS // 2


def _sc_agg_body(hp, col2d, row2d, q, cidx_v, ridx_v, rows_v, zblk_v,
                 acc_sh, gsem, ssem):
    c = lax.axis_index("c")
    s = lax.axis_index("s")
    w = c * NS + s

    z16 = jnp.zeros((16,), jnp.float32)
    for r in range(16):
        for cc in range(D // 16):
            zblk_v[r, pl.ds(cc * 16, 16)] = z16

    def _zero(k, carry):
        pltpu.sync_copy(zblk_v, acc_sh.at[pl.ds(s * 640 + k * 16, 16), :])
        return carry

    lax.fori_loop(0, 40, _zero, 0, unroll=False)

    b0 = w * _AGG_ROWS
    plsc.subcore_barrier()

    for h in range(2):
        pltpu.sync_copy(
            col2d.at[pl.ds(b0 + h * _HALF, _HALF), :], cidx_v)
        pltpu.sync_copy(
            row2d.at[pl.ds(b0 + h * _HALF, _HALF), :], ridx_v)
        pltpu.async_copy(hp.at[cidx_v.at[0]], rows_v.at[0], gsem)

        def _agg2(jj, carry):
            for b in range(2):
                j = jj * 2 + b
                pltpu.make_async_copy(
                    hp.at[cidx_v.at[j]], rows_v.at[b], gsem).wait()
                pltpu.async_copy(
                    rows_v.at[b], acc_sh.at[ridx_v.at[j]], ssem, add=True)

                @pl.when(j >= 1)
                def _():
                    pltpu.make_async_copy(
                        rows_v.at[b], acc_sh.at[ridx_v.at[j]], ssem).wait()

                @pl.when(j + 1 < _HALF)
                def _():
                    pltpu.async_copy(
                        hp.at[cidx_v.at[j + 1]], rows_v.at[(b + 1) % 2],
                        gsem)
            return carry

        lax.fori_loop(0, _HALF // 2, _agg2, 0, unroll=False)

        pltpu.make_async_copy(
            rows_v.at[0], acc_sh.at[ridx_v.at[0]], ssem).wait()

    plsc.subcore_barrier()

    pltpu.sync_copy(acc_sh.at[pl.ds(s * 640, 640), :],
                    q.at[pl.ds(c * NPAD + s * 640, 640), :])


_sc_agg = functools.partial(
    pl.kernel,
    out_type=jax.ShapeDtypeStruct((NC * NPAD, D), jnp.float32),
    mesh=_MESH,
    scratch_types=[
        pltpu.VMEM((_HALF, CH), jnp.int32),
        pltpu.VMEM((_HALF, CH), jnp.int32),
        pltpu.VMEM((2, CH, D), jnp.float32),
        pltpu.VMEM((16, D), jnp.float32),
        pltpu.VMEM_SHARED((NPAD, D), jnp.float32),
        pltpu.SemaphoreType.DMA,
        pltpu.SemaphoreType.DMA,
    ],
)(_sc_agg_body)


def _tc_comb_body(self_ref, dis_ref, q0_ref, q1_ref, o_ref):
    o_ref[...] = self_ref[...] + dis_ref[...] * (q0_ref[...] + q1_ref[...])


_tc_comb = pl.pallas_call(
    _tc_comb_body,
    out_shape=jax.ShapeDtypeStruct((N, D), jnp.float32),
)


def kernel(x, edge_index, self_weight, neighbor_weight, bias):
    npad_edges = ERP * D - E
    row_pad = jnp.concatenate(
        [edge_index[0], jnp.full((npad_edges,), NPAD - 1, jnp.int32)])
    col_pad = jnp.concatenate(
        [edge_index[1], jnp.zeros((npad_edges,), jnp.int32)])
    row2d = row_pad.reshape(ERP, D)
    col2d = col_pad.reshape(ERP, D)
    dis128 = _sc_deg(row2d)
    dis_n = dis128[:N]
    selfs, hp = _tc_main(x, self_weight, neighbor_weight,
                         bias.reshape(1, D), dis_n)
    q = _sc_agg(hp, col2d, row2d)
    return _tc_comb(selfs, dis_n, q[:N], q[NPAD:NPAD + N])


# speedup vs baseline: 17.0305x; 1.4277x over previous
import functools

import jax
import jax.numpy as jnp
from jax import lax
from jax.experimental import pallas as pl
from jax.experimental.pallas import tpu as pltpu
from jax.experimental.pallas import tpu_sc as plsc

N = 10000
E = 320000
D = 128
DH = D // 2
NPAD = 10240
NC = 2
NS = 16
ER = E // D
ERP = 2560
CH = 128

_MESH = plsc.VectorSubcoreMesh(
    core_axis_name="c", subcore_axis_name="s", num_cores=NC, num_subcores=NS)

_DEG_ROWS = ERP // NS
_NODES_PER_SUB = NPAD // (NC * NS)


def _sc_deg_body(row2d, dis128, idx_v, ones_v, zro_v, dstage_v, drows_v,
                 deg_sh, sem):
    c = lax.axis_index("c")
    s = lax.axis_index("s")

    z16 = jnp.zeros((16,), jnp.float32)
    for k in range(640 // 16):
        zro_v[pl.ds(k * 16, 16)] = z16
    o16 = jnp.ones((16,), jnp.float32)
    for k in range(CH // 16):
        ones_v[pl.ds(k * 16, 16)] = o16

    pltpu.sync_copy(zro_v, deg_sh.at[pl.ds(s * 640, 640)])

    pltpu.sync_copy(row2d.at[pl.ds(s * _DEG_ROWS, _DEG_ROWS), :], idx_v)

    plsc.subcore_barrier()

    def _scatter(j, carry):
        pltpu.sync_copy(ones_v, deg_sh.at[idx_v.at[j]], add=True)
        return carry

    lax.fori_loop(0, _DEG_ROWS, _scatter, 0, unroll=False)

    plsc.subcore_barrier()

    nbase = (c * NS + s) * _NODES_PER_SUB
    pltpu.sync_copy(deg_sh.at[pl.ds(nbase, _NODES_PER_SUB)], dstage_v)

    for v in range(_NODES_PER_SUB // 16):
        d = dstage_v[pl.ds(v * 16, 16)]
        i = lax.bitcast_convert_type(d, jnp.int32)
        i = jnp.int32(0x5F3759DF) - lax.shift_right_logical(i, 1)
        y = lax.bitcast_convert_type(i, jnp.float32)
        half = d * jnp.float32(0.5)
        for _ in range(3):
            y = y * (jnp.float32(1.5) - half * y * y)
        dstage_v[pl.ds(v * 16, 16)] = y

    def _splat(g, carry):
        v = dstage_v[pl.ds(g * 16, 16)]
        for r in range(16):
            v16 = lax.broadcast(v[r], (16,))
            for cc in range(D // 16):
                drows_v[r, pl.ds(cc * 16, 16)] = v16
        pltpu.sync_copy(drows_v, dis128.at[pl.ds(nbase + g * 16, 16), :])
        return carry

    lax.fori_loop(0, _NODES_PER_SUB // 16, _splat, 0, unroll=False)


_sc_deg = functools.partial(
    pl.kernel,
    out_type=jax.ShapeDtypeStruct((NPAD, D), jnp.float32),
    mesh=_MESH,
    scratch_types=[
        pltpu.VMEM((_DEG_ROWS, CH), jnp.int32),
        pltpu.VMEM((CH,), jnp.float32),
        pltpu.VMEM((640,), jnp.float32),
        pltpu.VMEM((_NODES_PER_SUB,), jnp.float32),
        pltpu.VMEM((16, D), jnp.float32),
        pltpu.VMEM_SHARED((NPAD,), jnp.float32),
        pltpu.SemaphoreType.DMA,
    ],
)(_sc_deg_body)


def _tc_main_body(x_ref, ws_ref, wn_ref, b_ref, dis_ref, self_ref, hps_ref):
    x = x_ref[...]
    self_ref[...] = (
        jnp.dot(x, ws_ref[...], preferred_element_type=jnp.float32)
        + b_ref[...])
    hp = dis_ref[...] * jnp.dot(
        x, wn_ref[...], preferred_element_type=jnp.float32)
    hps_ref[0] = hp[:, :DH]
    hps_ref[1] = hp[:, DH:]


_tc_main = pl.pallas_call(
    _tc_main_body,
    out_shape=(
        jax.ShapeDtypeStruct((N, D), jnp.float32),
        jax.ShapeDtypeStruct((NC, N, DH), jnp.float32),
    ),
)


_AGG_ROWS = ERP // NS


def _sc_agg_body(hps, col2d, row2d, q, cidx_v, ridx_v, rows_v, zblk_v,
                 acc_sh, gsem, ssem):
    c = lax.axis_index("c")
    s = lax.axis_index("s")

    z16 = jnp.zeros((16,), jnp.float32)
    for r in range(16):
        for cc in range(DH // 16):
            zblk_v[r, pl.ds(cc * 16, 16)] = z16

    def _zero(k, carry):
        pltpu.sync_copy(zblk_v, acc_sh.at[pl.ds(s * 640 + k * 16, 16), :])
        return carry

    lax.fori_loop(0, 40, _zero, 0, unroll=False)

    b0 = s * _AGG_ROWS
    pltpu.sync_copy(col2d.at[pl.ds(b0, _AGG_ROWS), :], cidx_v)
    pltpu.sync_copy(row2d.at[pl.ds(b0, _AGG_ROWS), :], ridx_v)

    plsc.subcore_barrier()

    hpc = hps.at[c]

    for j0 in range(3):
        pltpu.async_copy(hpc.at[cidx_v.at[j0]], rows_v.at[j0], gsem)

    def _agg4(jj, carry):
        for b in range(4):
            j = jj * 4 + b
            pltpu.make_async_copy(
                hpc.at[cidx_v.at[j]], rows_v.at[b], gsem).wait()
            pltpu.async_copy(
                rows_v.at[b], acc_sh.at[ridx_v.at[j]], ssem, add=True)

            @pl.when(j >= 1)
            def _():
                pltpu.make_async_copy(
                    rows_v.at[b], acc_sh.at[ridx_v.at[j]], ssem).wait()

            @pl.when(j + 3 < _AGG_ROWS)
            def _():
                pltpu.async_copy(
                    hpc.at[cidx_v.at[j + 3]], rows_v.at[(b + 3) % 4], gsem)
        return carry

    lax.fori_loop(0, _AGG_ROWS // 4, _agg4, 0, unroll=False)

    pltpu.make_async_copy(
        rows_v.at[0], acc_sh.at[ridx_v.at[0]], ssem).wait()

    plsc.subcore_barrier()

    pltpu.sync_copy(acc_sh.at[pl.ds(s * 640, 640), :],
                    q.at[pl.ds(c * NPAD + s * 640, 640), :])


_sc_agg = functools.partial(
    pl.kernel,
    out_type=jax.ShapeDtypeStruct((NC * NPAD, DH), jnp.float32),
    mesh=_MESH,
    compiler_params=pltpu.CompilerParams(use_tc_tiling_on_sc=False),
    scratch_types=[
        pltpu.VMEM((_AGG_ROWS, CH), jnp.int32),
        pltpu.VMEM((_AGG_ROWS, CH), jnp.int32),
        pltpu.VMEM((4, CH, DH), jnp.float32),
        pltpu.VMEM((16, DH), jnp.float32),
        pltpu.VMEM_SHARED((NPAD, DH), jnp.float32),
        pltpu.SemaphoreType.DMA,
        pltpu.SemaphoreType.DMA,
    ],
)(_sc_agg_body)


def _tc_comb_body(self_ref, dis_ref, q0_ref, q1_ref, o_ref):
    acc = jnp.concatenate([q0_ref[...], q1_ref[...]], axis=1)
    o_ref[...] = self_ref[...] + dis_ref[...] * acc


_tc_comb = pl.pallas_call(
    _tc_comb_body,
    out_shape=jax.ShapeDtypeStruct((N, D), jnp.float32),
)


def kernel(x, edge_index, self_weight, neighbor_weight, bias):
    npad_edges = ERP * D - E
    row_pad = jnp.concatenate(
        [edge_index[0], jnp.full((npad_edges,), NPAD - 1, jnp.int32)])
    col_pad = jnp.concatenate(
        [edge_index[1], jnp.zeros((npad_edges,), jnp.int32)])
    row2d = row_pad.reshape(ERP, D)
    col2d = col_pad.reshape(ERP, D)
    dis128 = _sc_deg(row2d)
    dis_n = dis128[:N]
    selfs, hps = _tc_main(x, self_weight, neighbor_weight,
                          bias.reshape(1, D), dis_n)
    q = _sc_agg(hps, col2d, row2d)
    return _tc_comb(selfs, dis_n, q[:N], q[NPAD:NPAD + N])
